# baseline (speedup 1.0000x reference)
import functools

import jax
import jax.numpy as jnp
from jax import lax
from jax.experimental import pallas as pl
from jax.experimental.pallas import tpu as pltpu
from jax.experimental.pallas import tpu_sc as plsc


_K = 9
_S = 1 << _K
_V = 10
_N = 18
_NC = 2
_NS = 16
_NW = _NC * _NS
_L = 16
_APW = _S // _NW
_NV = _S // _L
_PR = 248

_f32 = jnp.float32


def _gtake(x, idx):
    return x.at[idx].get(mode="promise_in_bounds")


def _bsum(x, lanes):
    for st in (1, 2, 4, 8):
        x = x + _gtake(x, lanes ^ st)
    return x


def _bcast(x, k):
    return _gtake(x, jnp.full((_L,), k, jnp.int32))


def _sc_body(params_hbm, out_hbm, pv, bcol, ebv, cacc, obuf):
    c = lax.axis_index("c")
    s = lax.axis_index("s")
    wid = s * _NC + c
    a_base = wid * _APW

    pltpu.sync_copy(params_hbm, pv)

    lanes = lax.iota(jnp.int32, _L)
    zero = jnp.zeros((_L,), _f32)

    def build_v(v, carry):
        bvec = lanes + v * _L
        bf = []
        for j in range(_K):
            bj = jnp.where(((bvec >> j) & 1) == 1, 1.0, 0.0).astype(_f32)
            bcol[j, pl.ds(v * _L, _L)] = bj
            bf.append(bj)
        eb = zero
        for i in range(_K):
            hi = zero
            for j in range(_K):
                hi = hi + pv[162 + i * _K + j] * bf[j]
            eb = eb + bf[i] * hi
        ebv[pl.ds(v * _L, _L)] = eb
        cacc[pl.ds(v * _L, _L)] = zero
        return carry

    lax.fori_loop(0, _NV, build_v, 0)

    avec = a_base + lanes
    af = [jnp.where(((avec >> i) & 1) == 1, 1.0, 0.0).astype(_f32)
          for i in range(_K)]
    ea_v = zero
    for i in range(_K):
        hi = zero
        for j in range(_K):
            hi = hi + pv[81 + i * _K + j] * af[j]
        ea_v = ea_v + af[i] * hi
    g_v = []
    for j in range(_K):
        gj = zero
        for i in range(_K):
            gj = gj + pv[i * _K + j] * af[i]
        g_v.append(gj)

    plo = zero
    zvec = zero
    for al in range(_APW):
        ea_b = _bcast(ea_v, al)
        gb = [_bcast(g_v[j], al) for j in range(_K)]

        def inner(v, rs):
            t = ebv[pl.ds(v * _L, _L)] + ea_b
            for j in range(_K):
                t = t + gb[j] * bcol[j, pl.ds(v * _L, _L)]
            w = jnp.exp(t)
            cacc[pl.ds(v * _L, _L)] = cacc[pl.ds(v * _L, _L)] + w
            return rs + w

        rs = lax.fori_loop(0, _NV, inner, zero)
        rsb = _bsum(rs, lanes)
        zvec = zvec + rsb
        a_full = jnp.full((_L,), a_base + al, jnp.int32)
        a_bits = jnp.where(((a_full >> lanes) & 1) == 1, 1.0, 0.0).astype(_f32)
        plo = plo + rsb * a_bits

    def philoop(v, acc):
        cv = cacc[pl.ds(v * _L, _L)]
        return tuple(acc[j] + cv * bcol[j, pl.ds(v * _L, _L)]
                     for j in range(_K))

    phiacc = lax.fori_loop(0, _NV, philoop, tuple(zero for _ in range(_K)))
    phi = zero
    for j in range(_K):
        oh = jnp.where(lanes == j, 1.0, 0.0).astype(_f32)
        phi = phi + _bsum(phiacc[j], lanes) * oh

    obuf[pl.ds(0, _L)] = plo
    obuf[pl.ds(_L, _L)] = phi
    obuf[pl.ds(2 * _L, _L)] = zvec
    pltpu.sync_copy(obuf, out_hbm.at[wid])


_mesh = plsc.VectorSubcoreMesh(core_axis_name="c", subcore_axis_name="s",
                               num_cores=_NC, num_subcores=_NS)

_sc_call = functools.partial(
    pl.kernel,
    out_type=jax.ShapeDtypeStruct((_NW, 3 * _L), _f32),
    mesh=_mesh,
    scratch_types=[
        pltpu.VMEM((_PR, _L), _f32),
        pltpu.VMEM((_K, _S), _f32),
        pltpu.VMEM((_S,), _f32),
        pltpu.VMEM((_S,), _f32),
        pltpu.VMEM((3 * _L,), _f32),
    ],
)(_sc_body)


def kernel(matrix, beta):
    m = beta * matrix.astype(_f32)
    flat = jnp.concatenate([
        (2.0 * m[:_K, _K:]).reshape(_K * _K),
        m[:_K, :_K].reshape(_K * _K),
        m[_K:, _K:].reshape(_K * _K),
        jnp.zeros((_PR - 3 * _K * _K,), _f32),
    ])
    pb = jnp.broadcast_to(flat[:, None], (_PR, _L))
    out = _sc_call(pb)
    plo = jnp.sum(out[:, :_L], axis=0)
    phi = jnp.sum(out[:, _L:2 * _L], axis=0)
    z = jnp.sum(out[:, 2 * _L])
    prob = jnp.concatenate([plo[:_K], phi[:_K]]) / z
    return prob[None, :_V], prob[None, _V:_N]


# speedup vs baseline: 1.3138x; 1.3138x over previous
import functools

import jax
import jax.numpy as jnp
from jax import lax
from jax.experimental import pallas as pl
from jax.experimental.pallas import tpu as pltpu
from jax.experimental.pallas import tpu_sc as plsc


_K = 9
_S = 1 << _K
_V = 10
_N = 18
_NC = 2
_NS = 16
_NW = _NC * _NS
_L = 16
_AG = 8
_BG = 4
_APW = _S // _AG
_BPW = _S // _BG
_NVW = _BPW // _L
_PR = 248

_f32 = jnp.float32
_i32 = jnp.int32


def _tree(terms):
    terms = list(terms)
    while len(terms) > 1:
        nxt = [terms[i] + terms[i + 1] for i in range(0, len(terms) - 1, 2)]
        if len(terms) % 2:
            nxt.append(terms[-1])
        terms = nxt
    return terms[0]


def _gtake(x, idx):
    return x.at[idx].get(mode="promise_in_bounds")


def _bsum(x, lanes):
    for st in (1, 2, 4, 8):
        x = x + _gtake(x, lanes ^ st)
    return x


def _bits(vec):
    return [jnp.where(((vec >> j) & 1) == 1, 1.0, 0.0).astype(_f32)
            for j in range(_K)]


def _quadform(pv, base, bf):
    return _tree([bf[i] * _tree([pv[base + i * _K + j] * bf[j]
                                 for j in range(_K)])
                  for i in range(_K)])


def _sc_body(params_hbm, out_hbm, pv, bcol, ebv, cacc, eat, gtab, obuf):
    c = lax.axis_index("c")
    s = lax.axis_index("s")
    wid = s * _NC + c
    ag = wid // _BG
    bg = wid - ag * _BG
    a_base = ag * _APW
    b_base = bg * _BPW

    pltpu.sync_copy(params_hbm, pv)

    lanes = lax.iota(_i32, _L)
    zero = jnp.zeros((_L,), _f32)

    def build_v(v, carry):
        bvec = b_base + v * _L + lanes
        bf = _bits(bvec)
        for j in range(_K):
            bcol[j, pl.ds(v * _L, _L)] = bf[j]
        ebv[pl.ds(v * _L, _L)] = _quadform(pv, 162, bf)
        cacc[pl.ds(v * _L, _L)] = zero
        return carry

    lax.fori_loop(0, _NVW, build_v, 0)

    def build_a(g, carry):
        avec = a_base + g * _L + lanes
        af = _bits(avec)
        eat[pl.ds(g * _L, _L)] = _quadform(pv, 81, af)
        for j in range(_K):
            gtab[j, pl.ds(g * _L, _L)] = _tree(
                [pv[i * _K + j] * af[i] for i in range(_K)])
        return carry

    lax.fori_loop(0, _APW // _L, build_a, 0)

    def arow(al, carry):
        plo, zvec = carry
        grp = (al // _L) * _L
        off = jnp.full((_L,), al - grp, _i32)
        ea_b = _gtake(eat[pl.ds(grp, _L)], off)
        gb = [_gtake(gtab[j, pl.ds(grp, _L)], off) for j in range(_K)]
        rs0 = zero
        rs1 = zero
        for u in range(_NVW):
            t = _tree([ebv[pl.ds(u * _L, _L)] + ea_b]
                      + [gb[j] * bcol[j, pl.ds(u * _L, _L)]
                         for j in range(_K)])
            w = jnp.exp(t)
            cacc[pl.ds(u * _L, _L)] = cacc[pl.ds(u * _L, _L)] + w
            if u % 2 == 0:
                rs0 = rs0 + w
            else:
                rs1 = rs1 + w
        rsb = _bsum(rs0 + rs1, lanes)
        zvec = zvec + rsb
        a_full = jnp.full((_L,), a_base + al, _i32)
        a_bits = jnp.where(((a_full >> lanes) & 1) == 1, 1.0, 0.0).astype(_f32)
        return plo + rsb * a_bits, zvec

    plo, zvec = lax.fori_loop(0, _APW, arow, (zero, zero))

    def philoop(v, acc):
        cv = cacc[pl.ds(v * _L, _L)]
        return tuple(acc[j] + cv * bcol[j, pl.ds(v * _L, _L)]
                     for j in range(_K))

    phiacc = lax.fori_loop(0, _NVW, philoop, tuple(zero for _ in range(_K)))
    phi = zero
    for j in range(_K):
        oh = jnp.where(lanes == j, 1.0, 0.0).astype(_f32)
        phi = phi + _bsum(phiacc[j], lanes) * oh

    obuf[pl.ds(0, _L)] = plo
    obuf[pl.ds(_L, _L)] = phi
    obuf[pl.ds(2 * _L, _L)] = zvec
    pltpu.sync_copy(obuf, out_hbm.at[wid])


_mesh = plsc.VectorSubcoreMesh(core_axis_name="c", subcore_axis_name="s",
                               num_cores=_NC, num_subcores=_NS)

_sc_call = functools.partial(
    pl.kernel,
    out_type=jax.ShapeDtypeStruct((_NW, 3 * _L), _f32),
    mesh=_mesh,
    scratch_types=[
        pltpu.VMEM((_PR, _L), _f32),
        pltpu.VMEM((_K, _BPW), _f32),
        pltpu.VMEM((_BPW,), _f32),
        pltpu.VMEM((_BPW,), _f32),
        pltpu.VMEM((_APW,), _f32),
        pltpu.VMEM((_K, _APW), _f32),
        pltpu.VMEM((3 * _L,), _f32),
    ],
)(_sc_body)


def kernel(matrix, beta):
    m = beta * matrix.astype(_f32)
    flat = jnp.concatenate([
        (2.0 * m[:_K, _K:]).reshape(_K * _K),
        m[:_K, :_K].reshape(_K * _K),
        m[_K:, _K:].reshape(_K * _K),
        jnp.zeros((_PR - 3 * _K * _K,), _f32),
    ])
    pb = jnp.broadcast_to(flat[:, None], (_PR, _L))
    out = _sc_call(pb)
    plo = jnp.sum(out[:, :_L], axis=0)
    phi = jnp.sum(out[:, _L:2 * _L], axis=0)
    z = jnp.sum(out[:, 2 * _L])
    prob = jnp.concatenate([plo[:_K], phi[:_K]]) / z
    return prob[None, :_V], prob[None, _V:_N]


# speedup vs baseline: 1.4289x; 1.0876x over previous
import functools

import jax
import jax.numpy as jnp
from jax import lax
from jax.experimental import pallas as pl
from jax.experimental.pallas import tpu as pltpu
from jax.experimental.pallas import tpu_sc as plsc


_K = 9
_S = 1 << _K
_V = 10
_N = 18
_NC = 2
_NS = 16
_NW = _NC * _NS
_L = 16
_AG = 8
_BG = 4
_APW = _S // _AG
_BPW = _S // _BG
_NVW = _BPW // _L
_PR = 248

_f32 = jnp.float32
_i32 = jnp.int32


def _tree(terms):
    terms = list(terms)
    while len(terms) > 1:
        nxt = [terms[i] + terms[i + 1] for i in range(0, len(terms) - 1, 2)]
        if len(terms) % 2:
            nxt.append(terms[-1])
        terms = nxt
    return terms[0]


def _gtake(x, idx):
    return x.at[idx].get(mode="promise_in_bounds")


def _bsum(x, lanes):
    for st in (1, 2, 4, 8):
        x = x + _gtake(x, lanes ^ st)
    return x


def _bits(vec):
    return [jnp.where(((vec >> j) & 1) == 1, 1.0, 0.0).astype(_f32)
            for j in range(_K)]


def _quadform(pv, base, bf):
    return _tree([bf[i] * _tree([pv[base + i * _K + j] * bf[j]
                                 for j in range(_K)])
                  for i in range(_K)])


def _sc_body(params_hbm, out_hbm, pv, ebv, cacc, eat, gtab, obuf):
    c = lax.axis_index("c")
    s = lax.axis_index("s")
    wid = s * _NC + c
    ag = wid // _BG
    bg = wid - ag * _BG
    a_base = ag * _APW
    b_base = bg * _BPW

    pltpu.sync_copy(params_hbm, pv)

    lanes = lax.iota(_i32, _L)
    zero = jnp.zeros((_L,), _f32)

    def _lmask(j):
        return jnp.where(((lanes >> j) & 1) == 1, 1.0, 0.0).astype(_f32)

    def _wmask(t):
        return jnp.where(((jnp.full((_L,), bg, _i32) >> t) & 1) == 1,
                         1.0, 0.0).astype(_f32)

    def build_v(v, carry):
        bvec = b_base + v * _L + lanes
        bf = _bits(bvec)
        ebv[pl.ds(v * _L, _L)] = _quadform(pv, 162, bf)
        cacc[pl.ds(v * _L, _L)] = zero
        return carry

    lax.fori_loop(0, _NVW, build_v, 0)

    def build_a(g, carry):
        avec = a_base + g * _L + lanes
        af = _bits(avec)
        eat[pl.ds(g * _L, _L)] = _quadform(pv, 81, af)
        for j in range(_K):
            gtab[j, pl.ds(g * _L, _L)] = _tree(
                [pv[i * _K + j] * af[i] for i in range(_K)])
        return carry

    lax.fori_loop(0, _APW // _L, build_a, 0)

    def arow(al, carry):
        plo, zvec = carry
        grp = (al // _L) * _L
        off = jnp.full((_L,), al - grp, _i32)
        ea_b = _gtake(eat[pl.ds(grp, _L)], off)
        gb = [_gtake(gtab[j, pl.ds(grp, _L)], off) for j in range(_K)]
        base = _tree([ea_b, _wmask(0) * gb[7], _wmask(1) * gb[8]])
        eac = [_tree([base]
                     + [gb[4 + t] for t in range(3) if (u >> t) & 1])
               for u in range(_NVW)]
        rs0 = zero
        rs1 = zero
        for u in range(_NVW):
            t = _tree([ebv[pl.ds(u * _L, _L)] + eac[u]]
                      + [_lmask(j) * gb[j] for j in range(4)])
            w = jnp.exp(t)
            cacc[pl.ds(u * _L, _L)] = cacc[pl.ds(u * _L, _L)] + w
            if u % 2 == 0:
                rs0 = rs0 + w
            else:
                rs1 = rs1 + w
        rsb = _bsum(rs0 + rs1, lanes)
        zvec = zvec + rsb
        a_full = jnp.full((_L,), a_base + al, _i32)
        a_bits = jnp.where(((a_full >> lanes) & 1) == 1, 1.0, 0.0).astype(_f32)
        return plo + rsb * a_bits, zvec

    plo, zvec = lax.fori_loop(0, _APW, arow, (zero, zero))

    cs = [cacc[pl.ds(u * _L, _L)] for u in range(_NVW)]
    call = _tree(cs)
    phi = zero
    for j in range(4):
        pj = _bsum(_lmask(j) * call, lanes)
        phi = phi + pj * jnp.where(lanes == j, 1.0, 0.0).astype(_f32)
    for t in range(3):
        sub = [cs[u] for u in range(_NVW) if (u >> t) & 1]
        pj = _bsum(_tree(sub), lanes)
        phi = phi + pj * jnp.where(lanes == 4 + t, 1.0, 0.0).astype(_f32)
    for t in range(2):
        pj = _bsum(_wmask(t) * call, lanes)
        phi = phi + pj * jnp.where(lanes == 7 + t, 1.0, 0.0).astype(_f32)

    obuf[pl.ds(0, _L)] = plo
    obuf[pl.ds(_L, _L)] = phi
    obuf[pl.ds(2 * _L, _L)] = zvec
    pltpu.sync_copy(obuf, out_hbm.at[wid])


_mesh = plsc.VectorSubcoreMesh(core_axis_name="c", subcore_axis_name="s",
                               num_cores=_NC, num_subcores=_NS)

_sc_call = functools.partial(
    pl.kernel,
    out_type=jax.ShapeDtypeStruct((_NW, 3 * _L), _f32),
    mesh=_mesh,
    scratch_types=[
        pltpu.VMEM((_PR, _L), _f32),
        pltpu.VMEM((_BPW,), _f32),
        pltpu.VMEM((_BPW,), _f32),
        pltpu.VMEM((_APW,), _f32),
        pltpu.VMEM((_K, _APW), _f32),
        pltpu.VMEM((3 * _L,), _f32),
    ],
)(_sc_body)


def kernel(matrix, beta):
    m = beta * matrix.astype(_f32)
    flat = jnp.concatenate([
        (2.0 * m[:_K, _K:]).reshape(_K * _K),
        m[:_K, :_K].reshape(_K * _K),
        m[_K:, _K:].reshape(_K * _K),
        jnp.zeros((_PR - 3 * _K * _K,), _f32),
    ])
    pb = jnp.broadcast_to(flat[:, None], (_PR, _L))
    out = _sc_call(pb)
    plo = jnp.sum(out[:, :_L], axis=0)
    phi = jnp.sum(out[:, _L:2 * _L], axis=0)
    z = jnp.sum(out[:, 2 * _L])
    prob = jnp.concatenate([plo[:_K], phi[:_K]]) / z
    return prob[None, :_V], prob[None, _V:_N]
